# baseline (speedup 1.0000x reference)
import functools

import jax
import jax.numpy as jnp
from jax import lax
from jax.experimental import pallas as pl
from jax.experimental.pallas import tpu as pltpu
from jax.experimental.pallas import tpu_sc as plsc

BATCH = 16384
INPUT_DIM = 128
EMBED_DIM = 64
NUM_DOMAINS = 100000

_INFO = plsc.get_sparse_core_info()
_NC, _NS = _INFO.num_cores, _INFO.num_subcores
_NW = _NC * _NS
_DIMS_PER_W = EMBED_DIM // _NW
_CHUNK = 4096
_NCHUNK = BATCH // _CHUNK
_UNROLL = 16


def _sc_gather_t(domain_ids, table_t):
    mesh = plsc.VectorSubcoreMesh(core_axis_name="c", subcore_axis_name="s")

    @functools.partial(
        pl.kernel,
        mesh=mesh,
        out_type=jax.ShapeDtypeStruct((EMBED_DIM, BATCH), jnp.float32),
        scratch_types=[
            pltpu.VMEM((NUM_DOMAINS,), jnp.float32),
            pltpu.VMEM((BATCH,), jnp.int32),
            pltpu.VMEM((2, _CHUNK), jnp.float32),
            pltpu.SemaphoreType.DMA,
            pltpu.SemaphoreType.DMA,
            pltpu.SemaphoreType.DMA,
        ],
        compiler_params=pltpu.CompilerParams(needs_layout_passes=False),
    )
    def gather_kernel(
        ids_hbm, table_hbm, out_hbm, slice_v, ids_v, out_v, s_ids, s_w0, s_w1
    ):
        wid = lax.axis_index("s") * _NC + lax.axis_index("c")
        cp_ids = pltpu.make_async_copy(ids_hbm, ids_v, s_ids)
        cp_ids.start()
        wsems = (s_w0, s_w1)
        pending = [None, None]

        def gather_dim(j):
            for k in range(_NCHUNK):
                buf = k % 2
                if pending[buf] is not None:
                    pending[buf].wait()

                @plsc.parallel_loop(0, _CHUNK // 16, unroll=_UNROLL)
                def body(c):
                    off = c * 16
                    idx = ids_v[pl.ds(k * _CHUNK + off, 16)]
                    out_v[buf, pl.ds(off, 16)] = plsc.load_gather(
                        slice_v, [idx]
                    )

                cp = pltpu.make_async_copy(
                    out_v.at[buf],
                    out_hbm.at[j, pl.ds(k * _CHUNK, _CHUNK)],
                    wsems[buf],
                )
                cp.start()
                pending[buf] = cp

        j0 = wid * _DIMS_PER_W
        pltpu.sync_copy(table_hbm.at[j0], slice_v)
        cp_ids.wait()
        gather_dim(j0)
        pltpu.sync_copy(table_hbm.at[j0 + 1], slice_v)
        gather_dim(j0 + 1)
        for cp in pending:
            cp.wait()

    return gather_kernel(domain_ids, table_t)


_BN = 4096
_BA = 16384


def _proj_body(w_ref, x_ref, b_ref, o_ref):
    proj = lax.dot_general(
        w_ref[...], x_ref[...],
        dimension_numbers=(((1,), (1,)), ((), ())),
        preferred_element_type=jnp.float32,
    )
    o_ref[...] = proj + b_ref[...]


def _tc_proj(W, x, b):
    grid = (BATCH // _BN,)
    return pl.pallas_call(
        _proj_body,
        grid=grid,
        in_specs=[
            pl.BlockSpec((EMBED_DIM, INPUT_DIM), lambda i: (0, 0)),
            pl.BlockSpec((_BN, INPUT_DIM), lambda i: (i, 0)),
            pl.BlockSpec((EMBED_DIM, 1), lambda i: (0, 0)),
        ],
        out_specs=pl.BlockSpec((EMBED_DIM, _BN), lambda i: (0, i)),
        out_shape=jax.ShapeDtypeStruct((EMBED_DIM, BATCH), jnp.float32),
    )(W, x, b)


def _add_body(p_ref, g_ref, o_ref):
    o_ref[...] = p_ref[...] + g_ref[...]


def _tc_add(p_t, g_t):
    grid = (BATCH // _BA,)
    return pl.pallas_call(
        _add_body,
        grid=grid,
        in_specs=[
            pl.BlockSpec((EMBED_DIM, _BA), lambda i: (0, i)),
            pl.BlockSpec((EMBED_DIM, _BA), lambda i: (0, i)),
        ],
        out_specs=pl.BlockSpec((EMBED_DIM, _BA), lambda i: (0, i)),
        out_shape=jax.ShapeDtypeStruct((EMBED_DIM, BATCH), jnp.float32),
    )(p_t, g_t)


@jax.jit
def kernel(x, domain_ids, domain_table, W, b):
    g_t = _sc_gather_t(domain_ids.astype(jnp.int32), domain_table.T)
    p_t = _tc_proj(W, x, b.reshape(EMBED_DIM, 1))
    out_t = _tc_add(p_t, g_t)
    return out_t.T


# speedup vs baseline: 1.0251x; 1.0251x over previous
import functools

import jax
import jax.numpy as jnp
from jax import lax
from jax.experimental import pallas as pl
from jax.experimental.pallas import tpu as pltpu
from jax.experimental.pallas import tpu_sc as plsc

BATCH = 16384
INPUT_DIM = 128
EMBED_DIM = 64
NUM_DOMAINS = 100000

_INFO = plsc.get_sparse_core_info()
_NC, _NS = _INFO.num_cores, _INFO.num_subcores
_NW = _NC * _NS
_DIMS_PER_W = EMBED_DIM // _NW
_CHUNK = 4096
_NCHUNK = BATCH // _CHUNK
_UNROLL = 16


def _sc_gather_t(domain_ids, table_t):
    mesh = plsc.VectorSubcoreMesh(core_axis_name="c", subcore_axis_name="s")

    @functools.partial(
        pl.kernel,
        mesh=mesh,
        out_type=jax.ShapeDtypeStruct((EMBED_DIM, BATCH), jnp.float32),
        scratch_types=[
            pltpu.VMEM((NUM_DOMAINS,), jnp.float32),
            pltpu.VMEM((BATCH,), jnp.int32),
            pltpu.VMEM((2, _CHUNK), jnp.float32),
            pltpu.SemaphoreType.DMA,
            pltpu.SemaphoreType.DMA,
            pltpu.SemaphoreType.DMA,
        ],
        compiler_params=pltpu.CompilerParams(needs_layout_passes=False),
    )
    def gather_kernel(
        ids_hbm, table_hbm, out_hbm, slice_v, ids_v, out_v, s_ids, s_w0, s_w1
    ):
        wid = lax.axis_index("s") * _NC + lax.axis_index("c")
        cp_ids = pltpu.make_async_copy(ids_hbm, ids_v, s_ids)
        cp_ids.start()
        wsems = (s_w0, s_w1)
        pending = [None, None]

        def gather_dim(j):
            for k in range(_NCHUNK):
                buf = k % 2
                if pending[buf] is not None:
                    pending[buf].wait()

                @plsc.parallel_loop(0, _CHUNK // 16, unroll=_UNROLL)
                def body(c):
                    off = c * 16
                    idx = ids_v[pl.ds(k * _CHUNK + off, 16)]
                    out_v[buf, pl.ds(off, 16)] = plsc.load_gather(
                        slice_v, [idx]
                    )

                cp = pltpu.make_async_copy(
                    out_v.at[buf],
                    out_hbm.at[j, pl.ds(k * _CHUNK, _CHUNK)],
                    wsems[buf],
                )
                cp.start()
                pending[buf] = cp

        j0 = wid * _DIMS_PER_W
        pltpu.sync_copy(table_hbm.at[j0], slice_v)
        cp_ids.wait()
        gather_dim(j0)
        pltpu.sync_copy(table_hbm.at[j0 + 1], slice_v)
        gather_dim(j0 + 1)
        for cp in pending:
            cp.wait()

    return gather_kernel(domain_ids, table_t)


_BN = 4096
_BA = 8192


def _proj_body(w_ref, x_ref, b_ref, o_ref):
    proj = lax.dot_general(
        w_ref[...], x_ref[...],
        dimension_numbers=(((1,), (1,)), ((), ())),
        preferred_element_type=jnp.float32,
    )
    o_ref[...] = proj + b_ref[...]


def _tc_proj(W, x, b):
    grid = (BATCH // _BN,)
    return pl.pallas_call(
        _proj_body,
        grid=grid,
        in_specs=[
            pl.BlockSpec((EMBED_DIM, INPUT_DIM), lambda i: (0, 0)),
            pl.BlockSpec((_BN, INPUT_DIM), lambda i: (i, 0)),
            pl.BlockSpec((EMBED_DIM, 1), lambda i: (0, 0)),
        ],
        out_specs=pl.BlockSpec((EMBED_DIM, _BN), lambda i: (0, i)),
        out_shape=jax.ShapeDtypeStruct((EMBED_DIM, BATCH), jnp.float32),
    )(W, x, b)


def _add_body(p_ref, g_ref, o_ref):
    o_ref[...] = p_ref[...] + g_ref[...]


def _tc_add(p_t, g_t):
    grid = (BATCH // _BA,)
    return pl.pallas_call(
        _add_body,
        grid=grid,
        in_specs=[
            pl.BlockSpec((EMBED_DIM, _BA), lambda i: (0, i)),
            pl.BlockSpec((EMBED_DIM, _BA), lambda i: (0, i)),
        ],
        out_specs=pl.BlockSpec((EMBED_DIM, _BA), lambda i: (0, i)),
        out_shape=jax.ShapeDtypeStruct((EMBED_DIM, BATCH), jnp.float32),
    )(p_t, g_t)


@jax.jit
def kernel(x, domain_ids, domain_table, W, b):
    g_t = _sc_gather_t(domain_ids.astype(jnp.int32), domain_table.T)
    p_t = _tc_proj(W, x, b.reshape(EMBED_DIM, 1))
    out_t = _tc_add(p_t, g_t)
    return out_t.T
